# Initial kernel scaffold; baseline (speedup 1.0000x reference)
import jax
import jax.numpy as jnp
from jax.experimental import pallas as pl


def kernel(coords, hash_table, offset_table, m0, m1):
    raise NotImplementedError("write your pallas kernel here")


# speedup vs baseline: 13.4277x; 13.4277x over previous
import functools

import jax
import jax.numpy as jnp
from jax import lax
from jax.experimental import pallas as pl
from jax.experimental.pallas import tpu as pltpu
from jax.experimental.pallas import tpu_sc as plsc

HASH_SIZE = 128
OFF_SIZE = 64
FEATS = 16
N_QUERIES = 1048576

NUM_WORKERS = 32
PER_WORKER = N_QUERIES // NUM_WORKERS
CHUNK = 2048
GATHERS = CHUNK // 128
N_CHUNKS = PER_WORKER // CHUNK

_mesh = plsc.VectorSubcoreMesh(core_axis_name="c", subcore_axis_name="s")


@functools.partial(
    pl.kernel,
    mesh=_mesh,
    compiler_params=pltpu.CompilerParams(use_tc_tiling_on_sc=False),
    out_type=jax.ShapeDtypeStruct((N_QUERIES, FEATS), jnp.float32),
    scratch_types=[
        pltpu.VMEM((CHUNK,), jnp.int32),
        pltpu.VMEM((CHUNK,), jnp.int32),
        pltpu.VMEM((CHUNK,), jnp.int32),
        pltpu.VMEM((CHUNK,), jnp.int32),
        pltpu.VMEM((CHUNK, FEATS), jnp.float32),
        pltpu.VMEM((GATHERS, 128), jnp.int32),
        pltpu.VMEM((GATHERS, 128), jnp.int32),
        pltpu.VMEM((3, 16), jnp.float32),
        pltpu.VMEM((3, 16), jnp.float32),
        pltpu.SemaphoreType.DMA,
    ],
)
def _psh_sc(coords_t_hbm, hashf_hbm, offp_hbm, m0_hbm, m1_hbm, out_hbm,
            c0_v, c1_v, c2_v, offw_v, feats_v, oidx_v, hidx_v, m0_v, m1_v,
            sem):
    wid = lax.axis_index("c") * 16 + lax.axis_index("s")
    base = wid * PER_WORKER
    pltpu.sync_copy(m0_hbm, m0_v)
    pltpu.sync_copy(m1_hbm, m1_v)
    cvs = (c0_v, c1_v, c2_v)

    def chunk_body(t, carry):
        row0 = base + t * CHUNK
        for d in range(3):
            pltpu.sync_copy(coords_t_hbm.at[pl.ds(d * N_QUERIES + row0, CHUNK)],
                            cvs[d])

        def pass_a(g, carry_a):
            for k in range(8):
                q0 = g * 128 + k * 16
                oi = []
                for d in range(3):
                    cd = cvs[d][pl.ds(q0, 16)]
                    cf = cd.astype(jnp.float32) * m1_v[d]
                    oi.append(cf.astype(jnp.int32) & (OFF_SIZE - 1))
                lin = (oi[0] << 12) | (oi[1] << 6) | oi[2]
                oidx_v[g, pl.ds(k * 16, 16)] = lin
            return carry_a

        lax.fori_loop(0, GATHERS, pass_a, 0)

        handles = [
            pltpu.async_copy(offp_hbm.at[oidx_v.at[g]],
                             offw_v.at[pl.ds(g * 128, 128)], sem)
            for g in range(GATHERS)
        ]
        for h in handles:
            h.wait()

        def pass_b(g, carry_b):
            for k in range(8):
                q0 = g * 128 + k * 16
                ow = offw_v[pl.ds(q0, 16)]
                od = (ow & 255, (ow >> 8) & 255, ow >> 16)
                hh = []
                for d in range(3):
                    cd = cvs[d][pl.ds(q0, 16)]
                    h0 = (cd.astype(jnp.float32) * m0_v[d]).astype(jnp.int32)
                    hh.append((h0 + od[d]) & (HASH_SIZE - 1))
                lin = (hh[0] << 14) | (hh[1] << 7) | hh[2]
                hidx_v[g, pl.ds(k * 16, 16)] = lin
            return carry_b

        lax.fori_loop(0, GATHERS, pass_b, 0)

        handles = [
            pltpu.async_copy(hashf_hbm.at[hidx_v.at[g]],
                             feats_v.at[pl.ds(g * 128, 128)], sem)
            for g in range(GATHERS)
        ]
        for h in handles:
            h.wait()

        pltpu.sync_copy(feats_v, out_hbm.at[pl.ds(row0, CHUNK)])
        return carry

    lax.fori_loop(0, N_CHUNKS, chunk_body, 0)


def kernel(coords, hash_table, offset_table, m0, m1):
    hashf = hash_table.reshape(HASH_SIZE ** 3, FEATS)
    off3 = offset_table.reshape(OFF_SIZE ** 3, 3)
    offp = off3[:, 0] | (off3[:, 1] << 8) | (off3[:, 2] << 16)
    coords_t = coords.T.reshape(-1)
    m0b = jnp.broadcast_to(m0.reshape(3, 1), (3, 16))
    m1b = jnp.broadcast_to(m1.reshape(3, 1), (3, 16))
    return _psh_sc(coords_t, hashf, offp, m0b, m1b)
